# baseline (speedup 1.0000x reference)
import jax
import jax.numpy as jnp
from jax.experimental import pallas as pl
from jax.experimental.pallas import tpu as pltpu

_B, _C, _D, _H, _W = 16, 4, 96, 96, 96
_BC = _B * _C
_ROWS = _D * _H
_PC = _BC // 2
_K = 4


def _loss_body(inp_hbm, lab_hbm, out_ref, bufs_i, bufs_l, sems_i, sems_l):
    c = pl.program_id(0)
    base = c * _PC

    def start(t, k):
        pltpu.make_async_copy(inp_hbm.at[base + t], bufs_i.at[k],
                              sems_i.at[k]).start()
        pltpu.make_async_copy(lab_hbm.at[base + t], bufs_l.at[k],
                              sems_l.at[k]).start()

    for k in range(_K):
        start(k, k)

    ssum = jnp.zeros((1, _W), jnp.float32)
    for t in range(_PC):
        k = t % _K
        pltpu.make_async_copy(bufs_i.at[k], bufs_i.at[k], sems_i.at[k]).wait()
        pltpu.make_async_copy(bufs_l.at[k], bufs_l.at[k], sems_l.at[k]).wait()
        x = bufs_i[k]
        lab = bufs_l[k]
        d = x - lab
        ssq = jnp.sum(d * d, axis=0, keepdims=True)
        sab = jnp.sum(jnp.abs(lab), axis=0, keepdims=True)
        if t + _K < _PC:
            start(t + _K, k)
        ssq_s = jnp.sum(ssq, axis=1, keepdims=True)
        sab_s = jnp.sum(sab, axis=1, keepdims=True)
        ssum = ssum + jnp.broadcast_to(ssq_s / sab_s, (1, _W))

    out_ref[...] = ssum.reshape(1, 1, _W) * (1.0 / (_B * _C))


def kernel(input, label):
    inp3 = input.reshape(_BC, _ROWS, _W)
    lab3 = label.reshape(_BC, _ROWS, _W)
    partials = pl.pallas_call(
        _loss_body,
        out_shape=jax.ShapeDtypeStruct((2, 1, _W), jnp.float32),
        grid=(2,),
        in_specs=[
            pl.BlockSpec(memory_space=pl.ANY),
            pl.BlockSpec(memory_space=pl.ANY),
        ],
        out_specs=pl.BlockSpec((1, 1, _W), lambda c: (c, 0, 0)),
        scratch_shapes=[
            pltpu.VMEM((_K, _ROWS, _W), jnp.float32),
            pltpu.VMEM((_K, _ROWS, _W), jnp.float32),
            pltpu.SemaphoreType.DMA((_K,)),
            pltpu.SemaphoreType.DMA((_K,)),
        ],
        compiler_params=pltpu.CompilerParams(
            dimension_semantics=("parallel",),
            vmem_limit_bytes=48 * 1024 * 1024,
        ),
        name="my_loss",
    )(inp3, lab3)
    loss = partials[0, 0, 0] + partials[1, 0, 0]
    return loss.reshape(1)


# speedup vs baseline: 1.2328x; 1.2328x over previous
import jax
import jax.numpy as jnp
from jax.experimental import pallas as pl
from jax.experimental.pallas import tpu as pltpu

_B, _C, _D, _H, _W = 16, 4, 96, 96, 96
_BC = _B * _C
_ROWS = _D * _H
_CPB = 2


def _loss_body(inp_ref, lab_ref, out_ref):
    x = inp_ref[...]
    lab = lab_ref[...]
    d = x - lab
    ssq = jnp.sum(d * d, axis=1)
    sab = jnp.sum(jnp.abs(lab), axis=1)
    ssq_s = jnp.sum(ssq, axis=1, keepdims=True)
    sab_s = jnp.sum(sab, axis=1, keepdims=True)
    ratio = ssq_s / sab_s
    out_ref[...] = jnp.broadcast_to(ratio[:, :, None], (_CPB, 1, _W))


def kernel(input, label):
    inp3 = input.reshape(_BC, _ROWS, _W)
    lab3 = label.reshape(_BC, _ROWS, _W)
    ratios = pl.pallas_call(
        _loss_body,
        out_shape=jax.ShapeDtypeStruct((_BC, 1, _W), jnp.float32),
        grid=(_BC // _CPB,),
        in_specs=[
            pl.BlockSpec((_CPB, _ROWS, _W), lambda i: (i, 0, 0)),
            pl.BlockSpec((_CPB, _ROWS, _W), lambda i: (i, 0, 0)),
        ],
        out_specs=pl.BlockSpec((_CPB, 1, _W), lambda i: (i, 0, 0)),
        compiler_params=pltpu.CompilerParams(
            dimension_semantics=("parallel",),
            vmem_limit_bytes=48 * 1024 * 1024,
        ),
        name="my_loss",
    )(inp3, lab3)
    loss = jnp.sum(ratios[:, 0, 0]) * (1.0 / (_B * _C))
    return loss.reshape(1)
